# baseline (speedup 1.0000x reference)
import functools

import jax
import jax.numpy as jnp
from jax import lax
from jax.experimental import pallas as pl
from jax.experimental.pallas import tpu as pltpu
from jax.experimental.pallas import tpu_sc as plsc

B = 256
T = 256
N = 256
L = 16
NG = N // L


K_SC = 144
SC_BPW_HI = 5


def _dominance_body(x_hbm, out_hbm, buf, mvec, vvec, rowbuf, found_ref, sem0, sem1):
    nc = 2
    wid = lax.axis_index("s") * nc + lax.axis_index("c")
    base = wid * 4 + jnp.minimum(wid, 16)
    nb = jnp.where(wid < 16, jnp.int32(SC_BPW_HI), jnp.int32(SC_BPW_HI - 1))

    half = T // 2
    buf0, buf1 = buf

    def start_h0(b):
        return pltpu.async_copy(x_hbm.at[b, pl.ds(0, half)], buf0, sem0)

    def start_h1(b):
        return pltpu.async_copy(x_hbm.at[b, pl.ds(half, half)], buf1, sem1)

    rowbuf[...] = jnp.zeros((L,), jnp.float32)
    start_h0(base)

    def per_batch_work(bi):
        b = base + bi
        start_h1(b)

        def make_acc(hbuf):
            def acc_body(ti, carry):
                new = list(carry)
                for u in range(2):
                    t = ti * 2 + u
                    for g in range(NG):
                        v = hbuf[t, pl.ds(g * L, L)]
                        new[2 * g] = new[2 * g] + v
                        new[2 * g + 1] = new[2 * g + 1] + v * v
                return tuple(new)

            return acc_body

        zeros = tuple(jnp.zeros((L,), jnp.float32) for _ in range(2 * NG))
        pltpu.make_async_copy(x_hbm.at[b, pl.ds(0, half)], buf0, sem0).wait()
        accs = lax.fori_loop(0, half // 2, make_acc(buf0), zeros)

        @pl.when(bi + 1 < nb)
        def _():
            start_h0(b + 1)

        pltpu.make_async_copy(x_hbm.at[b, pl.ds(half, half)], buf1, sem1).wait()
        accs = lax.fori_loop(0, half // 2, make_acc(buf1), accs)
        inv_t = jnp.float32(1.0 / T)
        for g in range(NG):
            mean = accs[2 * g] * inv_t
            var = accs[2 * g + 1] * inv_t - mean * mean
            mvec[pl.ds(g * L, L)] = mean
            vvec[pl.ds(g * L, L)] = var

        found_ref[0] = jnp.int32(0)

        def chunk_body(c, carry):
            @pl.when(found_ref[0] == 0)
            def _():
                mi_vec = mvec[pl.ds(c * L, L)]
                vi_vec = vvec[pl.ds(c * L, L)]
                mjs = [mvec[pl.ds(g * L, L)] for g in range(NG)]
                vjs = [vvec[pl.ds(g * L, L)] for g in range(NG)]
                hit = jnp.zeros((L,), jnp.bool_)
                for k in range(L):
                    mb = jnp.full((L,), mi_vec[k])
                    vb = jnp.full((L,), vi_vec[k])
                    for g in range(NG):
                        c_kg = jnp.logical_and(mjs[g] > mb, vjs[g] < vb)
                        hit = jnp.logical_or(hit, c_kg)
                hitf = jnp.where(hit, jnp.float32(1.0), jnp.float32(0.0))
                any_hit = hitf[0]
                for k in range(1, L):
                    any_hit = any_hit + hitf[k]
                found_ref[0] = jnp.where(any_hit > 0, jnp.int32(1), jnp.int32(0))

            return carry

        lax.fori_loop(0, NG, chunk_body, jnp.int32(0))

        keep = jnp.where(found_ref[0] > 0, jnp.float32(0.0), jnp.float32(1.0))
        lane = lax.iota(jnp.int32, L)
        rowbuf[...] = jnp.where(lane == bi, keep, rowbuf[...])

    def per_batch(bi, carry):
        @pl.when(bi < nb)
        def _():
            per_batch_work(bi)

        return carry

    lax.fori_loop(0, SC_BPW_HI, per_batch, jnp.int32(0))
    pltpu.sync_copy(rowbuf, out_hbm.at[wid])


@jax.jit
def _dominance_flags(x):
    mesh = plsc.VectorSubcoreMesh(core_axis_name="c", subcore_axis_name="s")
    fn = functools.partial(
        pl.kernel,
        out_type=jax.ShapeDtypeStruct((32, L), jnp.float32),
        mesh=mesh,
        scratch_types=[
            (pltpu.VMEM((T // 2, N), jnp.float32), pltpu.VMEM((T // 2, N), jnp.float32)),
            pltpu.VMEM((N + L,), jnp.float32),
            pltpu.VMEM((N + L,), jnp.float32),
            pltpu.VMEM((L,), jnp.float32),
            pltpu.SMEM((1,), jnp.int32),
            pltpu.SemaphoreType.DMA,
            pltpu.SemaphoreType.DMA,
        ],
    )(_dominance_body)
    return fn(x)


TC_BB = 8


def _tc_body(x_ref, out_ref):
    pid = pl.program_id(0)
    ones = jnp.ones((1, T), jnp.float32)
    inv_t = jnp.float32(1.0 / T)
    keeps = []
    for b in range(TC_BB):
        x = x_ref[b]
        s = jnp.dot(ones, x, preferred_element_type=jnp.float32)
        sq = jnp.dot(ones, x * x, preferred_element_type=jnp.float32)
        mean = s[0] * inv_t
        var = sq[0] * inv_t - mean * mean
        pair = jnp.logical_and(
            mean[None, :] > mean[:, None], var[None, :] < var[:, None]
        )
        found = jnp.any(pair)
        keeps.append(jnp.where(found, jnp.float32(0.0), jnp.float32(1.0)))
    cols = lax.broadcasted_iota(jnp.int32, (8, N), 1)
    prev = out_ref[...]
    v = jnp.where(pid == 0, jnp.zeros_like(prev), prev)
    for b in range(TC_BB):
        v = jnp.where(cols == pid * TC_BB + b, keeps[b], v)
    out_ref[...] = v


@jax.jit
def _tc_flags(x):
    nb = B - K_SC
    return pl.pallas_call(
        _tc_body,
        grid=(nb // TC_BB,),
        in_specs=[pl.BlockSpec((TC_BB, T, N), lambda b: (K_SC // TC_BB + b, 0, 0))],
        out_specs=pl.BlockSpec((8, N), lambda b: (0, 0)),
        out_shape=jax.ShapeDtypeStruct((8, N), jnp.float32),
    )(x)


def kernel(inputs):
    flags_sc = _dominance_flags(inputs)
    flags_tc = _tc_flags(inputs)
    keep_sc_hi = flags_sc[:16, :SC_BPW_HI].reshape(16 * SC_BPW_HI)
    keep_sc_lo = flags_sc[16:, : SC_BPW_HI - 1].reshape(16 * (SC_BPW_HI - 1))
    keep_tc = flags_tc[0, : B - K_SC]
    keep = jnp.concatenate([keep_sc_hi, keep_sc_lo, keep_tc])
    return jnp.broadcast_to(keep[None, :], (B, N))


# speedup vs baseline: 1.0744x; 1.0744x over previous
import functools

import jax
import jax.numpy as jnp
from jax import lax
from jax.experimental import pallas as pl
from jax.experimental.pallas import tpu as pltpu
from jax.experimental.pallas import tpu_sc as plsc

B = 256
T = 256
N = 256
L = 16
NG = N // L


K_SC = 128
SC_BPW = K_SC // 32


def _dominance_body(x_hbm, out_hbm, buf, mvec, vvec, rowbuf, found_ref, sem0, sem1):
    nc = 2
    wid = lax.axis_index("s") * nc + lax.axis_index("c")
    base = wid * SC_BPW
    nb = jnp.int32(SC_BPW)

    half = T // 2
    buf0, buf1 = buf

    def start_h0(b):
        return pltpu.async_copy(x_hbm.at[b, pl.ds(0, half)], buf0, sem0)

    def start_h1(b):
        return pltpu.async_copy(x_hbm.at[b, pl.ds(half, half)], buf1, sem1)

    rowbuf[...] = jnp.zeros((L,), jnp.float32)
    start_h0(base)

    def per_batch_work(bi):
        b = base + bi
        start_h1(b)

        def make_acc(hbuf):
            def acc_body(ti, carry):
                new = list(carry)
                for u in range(2):
                    t = ti * 2 + u
                    for g in range(NG):
                        v = hbuf[t, pl.ds(g * L, L)]
                        new[2 * g] = new[2 * g] + v
                        new[2 * g + 1] = new[2 * g + 1] + v * v
                return tuple(new)

            return acc_body

        zeros = tuple(jnp.zeros((L,), jnp.float32) for _ in range(2 * NG))
        pltpu.make_async_copy(x_hbm.at[b, pl.ds(0, half)], buf0, sem0).wait()
        accs = lax.fori_loop(0, half // 2, make_acc(buf0), zeros)

        @pl.when(bi + 1 < nb)
        def _():
            start_h0(b + 1)

        pltpu.make_async_copy(x_hbm.at[b, pl.ds(half, half)], buf1, sem1).wait()
        accs = lax.fori_loop(0, half // 2, make_acc(buf1), accs)
        inv_t = jnp.float32(1.0 / T)
        for g in range(NG):
            mean = accs[2 * g] * inv_t
            var = accs[2 * g + 1] * inv_t - mean * mean
            mvec[pl.ds(g * L, L)] = mean
            vvec[pl.ds(g * L, L)] = var

        found_ref[0] = jnp.int32(0)

        def chunk_body(c, carry):
            @pl.when(found_ref[0] == 0)
            def _():
                mi_vec = mvec[pl.ds(c * L, L)]
                vi_vec = vvec[pl.ds(c * L, L)]
                mjs = [mvec[pl.ds(g * L, L)] for g in range(NG)]
                vjs = [vvec[pl.ds(g * L, L)] for g in range(NG)]
                hit = jnp.zeros((L,), jnp.bool_)
                for k in range(L):
                    mb = jnp.full((L,), mi_vec[k])
                    vb = jnp.full((L,), vi_vec[k])
                    for g in range(NG):
                        c_kg = jnp.logical_and(mjs[g] > mb, vjs[g] < vb)
                        hit = jnp.logical_or(hit, c_kg)
                hitf = jnp.where(hit, jnp.float32(1.0), jnp.float32(0.0))
                any_hit = hitf[0]
                for k in range(1, L):
                    any_hit = any_hit + hitf[k]
                found_ref[0] = jnp.where(any_hit > 0, jnp.int32(1), jnp.int32(0))

            return carry

        lax.fori_loop(0, NG, chunk_body, jnp.int32(0))

        keep = jnp.where(found_ref[0] > 0, jnp.float32(0.0), jnp.float32(1.0))
        lane = lax.iota(jnp.int32, L)
        rowbuf[...] = jnp.where(lane == bi, keep, rowbuf[...])

    def per_batch(bi, carry):
        @pl.when(bi < nb)
        def _():
            per_batch_work(bi)

        return carry

    lax.fori_loop(0, SC_BPW, per_batch, jnp.int32(0))
    pltpu.sync_copy(rowbuf, out_hbm.at[wid])


@jax.jit
def _dominance_flags(x):
    mesh = plsc.VectorSubcoreMesh(core_axis_name="c", subcore_axis_name="s")
    fn = functools.partial(
        pl.kernel,
        out_type=jax.ShapeDtypeStruct((32, L), jnp.float32),
        mesh=mesh,
        scratch_types=[
            (pltpu.VMEM((T // 2, N), jnp.float32), pltpu.VMEM((T // 2, N), jnp.float32)),
            pltpu.VMEM((N + L,), jnp.float32),
            pltpu.VMEM((N + L,), jnp.float32),
            pltpu.VMEM((L,), jnp.float32),
            pltpu.SMEM((1,), jnp.int32),
            pltpu.SemaphoreType.DMA,
            pltpu.SemaphoreType.DMA,
        ],
    )(_dominance_body)
    return fn(x)


TC_BB = 16


def _tc_body(x_ref, out_ref):
    pid = pl.program_id(0)
    ones = jnp.ones((1, T), jnp.float32)
    inv_t = jnp.float32(1.0 / T)
    keeps = []
    for b in range(TC_BB):
        x = x_ref[b]
        s = jnp.dot(ones, x, preferred_element_type=jnp.float32)
        sq = jnp.dot(ones, x * x, preferred_element_type=jnp.float32)
        mean = s[0] * inv_t
        var = sq[0] * inv_t - mean * mean
        pair = jnp.logical_and(
            mean[None, :] > mean[:, None], var[None, :] < var[:, None]
        )
        found = jnp.any(pair)
        keeps.append(jnp.where(found, jnp.float32(0.0), jnp.float32(1.0)))
    cols = lax.broadcasted_iota(jnp.int32, (8, N), 1)
    prev = out_ref[...]
    v = jnp.where(pid == 0, jnp.zeros_like(prev), prev)
    for b in range(TC_BB):
        v = jnp.where(cols == pid * TC_BB + b, keeps[b], v)
    out_ref[...] = v


@jax.jit
def _tc_flags(x):
    nb = B - K_SC
    return pl.pallas_call(
        _tc_body,
        grid=(nb // TC_BB,),
        in_specs=[pl.BlockSpec((TC_BB, T, N), lambda b: (K_SC // TC_BB + b, 0, 0))],
        out_specs=pl.BlockSpec((8, N), lambda b: (0, 0)),
        out_shape=jax.ShapeDtypeStruct((8, N), jnp.float32),
    )(x)


def kernel(inputs):
    flags_sc = _dominance_flags(inputs)
    flags_tc = _tc_flags(inputs)
    keep_sc = flags_sc[:, :SC_BPW].reshape(K_SC)
    keep_tc = flags_tc[0, : B - K_SC]
    keep = jnp.concatenate([keep_sc, keep_tc])
    return jnp.broadcast_to(keep[None, :], (B, N))


# speedup vs baseline: 1.1028x; 1.0264x over previous
import functools

import jax
import jax.numpy as jnp
from jax import lax
from jax.experimental import pallas as pl
from jax.experimental.pallas import tpu as pltpu
from jax.experimental.pallas import tpu_sc as plsc

B = 256
T = 256
N = 256
L = 16
NG = N // L


K_SC = 128
SC_BPW = K_SC // 32


def _dominance_body(x_hbm, out_hbm, buf, mvec, vvec, rowbuf, found_ref, sem0, sem1):
    nc = 2
    wid = lax.axis_index("s") * nc + lax.axis_index("c")
    base = wid * SC_BPW
    nb = jnp.int32(SC_BPW)

    half = T // 2
    buf0, buf1 = buf

    def start_h0(b):
        return pltpu.async_copy(x_hbm.at[b, pl.ds(0, half)], buf0, sem0)

    def start_h1(b):
        return pltpu.async_copy(x_hbm.at[b, pl.ds(half, half)], buf1, sem1)

    rowbuf[...] = jnp.zeros((L,), jnp.float32)
    start_h0(base)

    def per_batch_work(bi):
        b = base + bi
        start_h1(b)

        def make_acc(hbuf):
            def acc_body(ti, carry):
                new = list(carry)
                for u in range(2):
                    t = ti * 2 + u
                    for g in range(NG):
                        v = hbuf[t, pl.ds(g * L, L)]
                        new[2 * g] = new[2 * g] + v
                        new[2 * g + 1] = new[2 * g + 1] + v * v
                return tuple(new)

            return acc_body

        zeros = tuple(jnp.zeros((L,), jnp.float32) for _ in range(2 * NG))
        pltpu.make_async_copy(x_hbm.at[b, pl.ds(0, half)], buf0, sem0).wait()
        accs = lax.fori_loop(0, half // 2, make_acc(buf0), zeros)

        @pl.when(bi + 1 < nb)
        def _():
            start_h0(b + 1)

        pltpu.make_async_copy(x_hbm.at[b, pl.ds(half, half)], buf1, sem1).wait()
        accs = lax.fori_loop(0, half // 2, make_acc(buf1), accs)
        inv_t = jnp.float32(1.0 / T)
        for g in range(NG):
            mean = accs[2 * g] * inv_t
            var = accs[2 * g + 1] * inv_t - mean * mean
            mvec[pl.ds(g * L, L)] = mean
            vvec[pl.ds(g * L, L)] = var

        found_ref[0] = jnp.int32(0)

        def chunk_body(c, carry):
            @pl.when(found_ref[0] == 0)
            def _():
                mi_vec = mvec[pl.ds(c * L, L)]
                vi_vec = vvec[pl.ds(c * L, L)]
                mjs = [mvec[pl.ds(g * L, L)] for g in range(NG)]
                vjs = [vvec[pl.ds(g * L, L)] for g in range(NG)]
                hit = jnp.zeros((L,), jnp.bool_)
                for k in range(L):
                    mb = jnp.full((L,), mi_vec[k])
                    vb = jnp.full((L,), vi_vec[k])
                    for g in range(NG):
                        c_kg = jnp.logical_and(mjs[g] > mb, vjs[g] < vb)
                        hit = jnp.logical_or(hit, c_kg)
                hitf = jnp.where(hit, jnp.float32(1.0), jnp.float32(0.0))
                any_hit = hitf[0]
                for k in range(1, L):
                    any_hit = any_hit + hitf[k]
                found_ref[0] = jnp.where(any_hit > 0, jnp.int32(1), jnp.int32(0))

            return carry

        lax.fori_loop(0, NG, chunk_body, jnp.int32(0))

        keep = jnp.where(found_ref[0] > 0, jnp.float32(0.0), jnp.float32(1.0))
        lane = lax.iota(jnp.int32, L)
        rowbuf[...] = jnp.where(lane == bi, keep, rowbuf[...])

    def per_batch(bi, carry):
        @pl.when(bi < nb)
        def _():
            per_batch_work(bi)

        return carry

    lax.fori_loop(0, SC_BPW, per_batch, jnp.int32(0))
    pltpu.sync_copy(rowbuf, out_hbm.at[wid])


@jax.jit
def _dominance_flags(x):
    mesh = plsc.VectorSubcoreMesh(core_axis_name="c", subcore_axis_name="s")
    fn = functools.partial(
        pl.kernel,
        out_type=jax.ShapeDtypeStruct((32, L), jnp.float32),
        mesh=mesh,
        scratch_types=[
            (pltpu.VMEM((T // 2, N), jnp.float32), pltpu.VMEM((T // 2, N), jnp.float32)),
            pltpu.VMEM((N + L,), jnp.float32),
            pltpu.VMEM((N + L,), jnp.float32),
            pltpu.VMEM((L,), jnp.float32),
            pltpu.SMEM((1,), jnp.int32),
            pltpu.SemaphoreType.DMA,
            pltpu.SemaphoreType.DMA,
        ],
    )(_dominance_body)
    return fn(x)


TC_BB = 32


def _tc_body(x_ref, out_ref):
    pid = pl.program_id(0)
    ones = jnp.ones((1, T), jnp.float32)
    inv_t = jnp.float32(1.0 / T)
    keeps = []
    for b in range(TC_BB):
        x = x_ref[b]
        s = jnp.dot(ones, x, preferred_element_type=jnp.float32)
        sq = jnp.dot(ones, x * x, preferred_element_type=jnp.float32)
        mean = s[0] * inv_t
        var = sq[0] * inv_t - mean * mean
        pair = jnp.logical_and(
            mean[None, :] > mean[:, None], var[None, :] < var[:, None]
        )
        found = jnp.any(pair)
        keeps.append(jnp.where(found, jnp.float32(0.0), jnp.float32(1.0)))
    cols = lax.broadcasted_iota(jnp.int32, (8, N), 1)
    prev = out_ref[...]
    v = jnp.where(pid == 0, jnp.zeros_like(prev), prev)
    for b in range(TC_BB):
        v = jnp.where(cols == pid * TC_BB + b, keeps[b], v)
    out_ref[...] = v


@jax.jit
def _tc_flags(x):
    nb = B - K_SC
    return pl.pallas_call(
        _tc_body,
        grid=(nb // TC_BB,),
        in_specs=[pl.BlockSpec((TC_BB, T, N), lambda b: (K_SC // TC_BB + b, 0, 0))],
        out_specs=pl.BlockSpec((8, N), lambda b: (0, 0)),
        out_shape=jax.ShapeDtypeStruct((8, N), jnp.float32),
    )(x)


def kernel(inputs):
    flags_sc = _dominance_flags(inputs)
    flags_tc = _tc_flags(inputs)
    keep_sc = flags_sc[:, :SC_BPW].reshape(K_SC)
    keep_tc = flags_tc[0, : B - K_SC]
    keep = jnp.concatenate([keep_sc, keep_tc])
    return jnp.broadcast_to(keep[None, :], (B, N))
